# baseline (speedup 1.0000x reference)
import functools

import jax
import jax.numpy as jnp
from jax import lax
from jax.experimental import pallas as pl
from jax.experimental.pallas import tpu as pltpu
from jax.experimental.pallas import tpu_sc as plsc

B, LF, LT, D, FS = 1024, 20, 30, 300, 2048
L = LF + LT
V = 100000
DP = 384
NTOK = B * LT
NC, NS = 2, 16
NW = NC * NS
ROWS_W = NTOK // NW
CHUNK = 120
NCH = ROWS_W // CHUNK

VB = 4096


def _tp_body(t_ref, o_ref):
    o_ref[:, :D] = jnp.transpose(t_ref[...])
    o_ref[:, D:] = jnp.zeros((o_ref.shape[0], DP - D), jnp.float32)


def _tc_transpose_pad(table_t):
    return pl.pallas_call(
        _tp_body,
        grid=(pl.cdiv(V, VB),),
        in_specs=[pl.BlockSpec((D, VB), lambda i: (0, i))],
        out_specs=pl.BlockSpec((VB, DP), lambda i: (i, 0)),
        out_shape=jax.ShapeDtypeStruct((V, DP), jnp.float32),
    )(table_t)


def _sc_gather(idx3, tab_rm):
    mesh = plsc.VectorSubcoreMesh(core_axis_name="c", subcore_axis_name="s")

    @functools.partial(
        pl.kernel,
        out_type=jax.ShapeDtypeStruct((NTOK, DP), jnp.float32),
        mesh=mesh,
        scratch_types=[
            pltpu.VMEM((NCH, CHUNK), jnp.int32),
            pltpu.VMEM((2, CHUNK, DP), jnp.float32),
            pltpu.SemaphoreType.DMA,
            pltpu.SemaphoreType.DMA,
        ],
    )
    def k(idx_hbm, tab_hbm, out_hbm, idx_v, buf_v, gsem, osem):
        wid = lax.axis_index("s") * NC + lax.axis_index("c")
        pltpu.sync_copy(idx_hbm.at[wid], idx_v)
        base = wid * ROWS_W

        def gather(c, slot):
            return pltpu.async_copy(
                tab_hbm.at[idx_v.at[c]], buf_v.at[slot], gsem)

        def put(c, slot):
            return pltpu.async_copy(
                buf_v.at[slot], out_hbm.at[pl.ds(base + c * CHUNK, CHUNK)],
                osem)

        gather(0, 0).wait()
        for c in range(NCH):
            if c + 1 < NCH:
                g = gather(c + 1, (c + 1) % 2)
            p = put(c, c % 2)
            if c + 1 < NCH:
                g.wait()
            p.wait()

    return k(idx3, tab_rm)


BBL = 1024


def _tail(emb, pos_ref, seg_ref, se_ref, g_ref, bt_ref, out_ref):
    emb = emb + pos_ref[...].reshape(D, 1)
    sg = seg_ref[...].reshape(1, BBL)
    se = se_ref[...]
    emb = emb + jnp.where(sg == 0, se[:, 0:1],
                          jnp.where(sg == 1, se[:, 1:2], se[:, 2:3]))
    mean = jnp.mean(emb, axis=0, keepdims=True)
    cen = emb - mean
    var = jnp.mean(cen * cen, axis=0, keepdims=True)
    out_ref[...] = ((cen * lax.rsqrt(var + 1e-5) * g_ref[...]
                     + bt_ref[...])[None])


def _tc_frames_body(frames_ref, w_ref, b_ref, pos_ref, seg_ref, se_ref,
                    g_ref, bt_ref, out_ref):
    f = frames_ref[...].astype(jnp.bfloat16).reshape(BBL, FS)
    emb = lax.dot_general(w_ref[...], f, (((1,), (1,)), ((), ())),
                          preferred_element_type=jnp.float32)
    _tail(emb + b_ref[...], pos_ref, seg_ref, se_ref, g_ref, bt_ref, out_ref)


def _tc_tok_body(prev_ref, tok_ref, pos_ref, seg_ref, se_ref, g_ref, bt_ref,
                 out_ref):
    del prev_ref
    t = jnp.transpose(tok_ref[...].reshape(BBL, DP))
    _tail(t[:D], pos_ref, seg_ref, se_ref, g_ref, bt_ref, out_ref)


def kernel(x, seg, frames_feature, tok_embed, pos_embed, seg_embed, W, b,
           gamma, beta):
    table_t = jnp.transpose(tok_embed)
    frames_t = jnp.transpose(frames_feature, (1, 0, 2))
    seg_t3 = jnp.transpose(seg).reshape(L, 1, B)
    idx3 = jnp.transpose(x).reshape(NW, NCH, CHUNK)
    w_t = jnp.transpose(W).astype(jnp.bfloat16)
    pos_3 = pos_embed[:L].reshape(L, D, 1)
    se_t = jnp.pad(jnp.transpose(seg_embed), ((0, 0), (0, 5)))
    b_c = b.reshape(D, 1)
    g_c = gamma.reshape(D, 1)
    bt_c = beta.reshape(D, 1)

    tab_rm = _tc_transpose_pad(table_t)
    tok = _sc_gather(idx3, tab_rm).reshape(LT, B, DP)

    out_f = pl.pallas_call(
        _tc_frames_body,
        grid=(LF, B // BBL),
        in_specs=[
            pl.BlockSpec((1, BBL, FS), lambda l, bb: (l, bb, 0)),
            pl.BlockSpec((D, FS), lambda l, bb: (0, 0)),
            pl.BlockSpec((D, 1), lambda l, bb: (0, 0)),
            pl.BlockSpec((1, D, 1), lambda l, bb: (l, 0, 0)),
            pl.BlockSpec((1, 1, BBL), lambda l, bb: (l, 0, bb)),
            pl.BlockSpec((D, 8), lambda l, bb: (0, 0)),
            pl.BlockSpec((D, 1), lambda l, bb: (0, 0)),
            pl.BlockSpec((D, 1), lambda l, bb: (0, 0)),
        ],
        out_specs=pl.BlockSpec((1, D, BBL), lambda l, bb: (l, 0, bb)),
        out_shape=jax.ShapeDtypeStruct((L, D, B), jnp.float32),
    )(frames_t, w_t, b_c, pos_3, seg_t3, se_t, g_c, bt_c)

    out_t = pl.pallas_call(
        _tc_tok_body,
        grid=(LT, B // BBL),
        in_specs=[
            pl.BlockSpec(memory_space=pltpu.MemorySpace.HBM),
            pl.BlockSpec((1, BBL, DP), lambda l, bb: (l, bb, 0)),
            pl.BlockSpec((1, D, 1), lambda l, bb: (l + LF, 0, 0)),
            pl.BlockSpec((1, 1, BBL), lambda l, bb: (l + LF, 0, bb)),
            pl.BlockSpec((D, 8), lambda l, bb: (0, 0)),
            pl.BlockSpec((D, 1), lambda l, bb: (0, 0)),
            pl.BlockSpec((D, 1), lambda l, bb: (0, 0)),
        ],
        out_specs=pl.BlockSpec((1, D, BBL), lambda l, bb: (l + LF, 0, bb)),
        out_shape=jax.ShapeDtypeStruct((L, D, B), jnp.float32),
        input_output_aliases={0: 0},
    )(out_f, tok, pos_3, seg_t3, se_t, g_c, bt_c)
    return jnp.transpose(out_t, (2, 0, 1))


# speedup vs baseline: 1.0070x; 1.0070x over previous
import functools

import jax
import jax.numpy as jnp
from jax import lax
from jax.experimental import pallas as pl
from jax.experimental.pallas import tpu as pltpu
from jax.experimental.pallas import tpu_sc as plsc

B, LF, LT, D, FS = 1024, 20, 30, 300, 2048
L = LF + LT
V = 100000
DP = 384
NTOK = B * LT
NC, NS = 2, 16
NW = NC * NS
ROWS_W = NTOK // NW
CHUNK = 120
NCH = ROWS_W // CHUNK

VB = 4096


def _tp_body(t_ref, o_ref):
    o_ref[:, :D] = jnp.transpose(t_ref[...])
    o_ref[:, D:] = jnp.zeros((o_ref.shape[0], DP - D), jnp.float32)


def _tc_transpose_pad(table_t):
    return pl.pallas_call(
        _tp_body,
        grid=(pl.cdiv(V, VB),),
        in_specs=[pl.BlockSpec((D, VB), lambda i: (0, i))],
        out_specs=pl.BlockSpec((VB, DP), lambda i: (i, 0)),
        out_shape=jax.ShapeDtypeStruct((V, DP), jnp.float32),
    )(table_t)


def _sc_gather(idx3, tab_rm):
    mesh = plsc.VectorSubcoreMesh(core_axis_name="c", subcore_axis_name="s")

    @functools.partial(
        pl.kernel,
        out_type=jax.ShapeDtypeStruct((NTOK, DP), jnp.float32),
        mesh=mesh,
        scratch_types=[
            pltpu.VMEM((NCH, CHUNK), jnp.int32),
            pltpu.VMEM((2, CHUNK, DP), jnp.float32),
            pltpu.SemaphoreType.DMA,
            pltpu.SemaphoreType.DMA,
        ],
    )
    def k(idx_hbm, tab_hbm, out_hbm, idx_v, buf_v, gsem, osem):
        wid = lax.axis_index("s") * NC + lax.axis_index("c")
        pltpu.sync_copy(idx_hbm.at[wid], idx_v)
        base = wid * ROWS_W

        def gather(c, slot):
            return pltpu.async_copy(
                tab_hbm.at[idx_v.at[c]], buf_v.at[slot], gsem)

        def put(c, slot):
            return pltpu.async_copy(
                buf_v.at[slot], out_hbm.at[pl.ds(base + c * CHUNK, CHUNK)],
                osem)

        gather(0, 0).wait()
        for c in range(NCH):
            if c + 1 < NCH:
                g = gather(c + 1, (c + 1) % 2)
            p = put(c, c % 2)
            if c + 1 < NCH:
                g.wait()
            p.wait()

    return k(idx3, tab_rm)


BBL = 1024


def _tc_body(frames_ref, w_ref, b_ref, tok_ref, pos_ref, seg_ref, se_ref,
             g_ref, bt_ref, out_ref):
    l = pl.program_id(0)

    def tail(emb):
        emb = emb + pos_ref[...].reshape(D, 1)
        sg = seg_ref[...].reshape(1, BBL)
        se = se_ref[...]
        emb = emb + jnp.where(sg == 0, se[:, 0:1],
                              jnp.where(sg == 1, se[:, 1:2], se[:, 2:3]))
        mean = jnp.mean(emb, axis=0, keepdims=True)
        cen = emb - mean
        var = jnp.mean(cen * cen, axis=0, keepdims=True)
        out_ref[...] = ((cen * lax.rsqrt(var + 1e-5) * g_ref[...]
                         + bt_ref[...])[None])

    @pl.when(l < LF)
    def _():
        f = frames_ref[...].astype(jnp.bfloat16).reshape(BBL, FS)
        emb = lax.dot_general(w_ref[...], f, (((1,), (1,)), ((), ())),
                              preferred_element_type=jnp.float32)
        tail(emb + b_ref[...])

    @pl.when(l >= LF)
    def _():
        t = jnp.transpose(tok_ref[...].reshape(BBL, DP))
        tail(t[:D])


def kernel(x, seg, frames_feature, tok_embed, pos_embed, seg_embed, W, b,
           gamma, beta):
    table_t = jnp.transpose(tok_embed)
    frames_t = jnp.transpose(frames_feature, (1, 0, 2))
    seg_t3 = jnp.transpose(seg).reshape(L, 1, B)
    idx3 = jnp.transpose(x).reshape(NW, NCH, CHUNK)
    w_t = jnp.transpose(W).astype(jnp.bfloat16)
    pos_3 = pos_embed[:L].reshape(L, D, 1)
    se_t = jnp.pad(jnp.transpose(seg_embed), ((0, 0), (0, 5)))
    b_c = b.reshape(D, 1)
    g_c = gamma.reshape(D, 1)
    bt_c = beta.reshape(D, 1)

    tab_rm = _tc_transpose_pad(table_t)
    tok = _sc_gather(idx3, tab_rm).reshape(LT, B, DP)

    out_t = pl.pallas_call(
        _tc_body,
        grid=(L, B // BBL),
        in_specs=[
            pl.BlockSpec((1, BBL, FS),
                         lambda l, bb: (jnp.minimum(l, LF - 1), bb, 0)),
            pl.BlockSpec((D, FS), lambda l, bb: (0, 0)),
            pl.BlockSpec((D, 1), lambda l, bb: (0, 0)),
            pl.BlockSpec((1, BBL, DP),
                         lambda l, bb: (jnp.maximum(l - LF, 0), bb, 0)),
            pl.BlockSpec((1, D, 1), lambda l, bb: (l, 0, 0)),
            pl.BlockSpec((1, 1, BBL), lambda l, bb: (l, 0, bb)),
            pl.BlockSpec((D, 8), lambda l, bb: (0, 0)),
            pl.BlockSpec((D, 1), lambda l, bb: (0, 0)),
            pl.BlockSpec((D, 1), lambda l, bb: (0, 0)),
        ],
        out_specs=pl.BlockSpec((1, D, BBL), lambda l, bb: (l, 0, bb)),
        out_shape=jax.ShapeDtypeStruct((L, D, B), jnp.float32),
    )(frames_t, w_t, b_c, tok, pos_3, seg_t3, se_t, g_c, bt_c)
    return jnp.transpose(out_t, (2, 0, 1))
